# baseline (speedup 1.0000x reference)
import functools

import jax
import jax.numpy as jnp
from jax import lax
from jax.experimental import pallas as pl
from jax.experimental.pallas import tpu as pltpu
from jax.experimental.pallas import tpu_sc as plsc

K = 16
BR = 256
RB = 16


def _dist_body(xr_ref, xt_ref, dist_ref, dst_ref, *, M, K):
    n = pl.program_id(0)
    rb = pl.program_id(1)
    xr = xr_ref[0]
    xt = xt_ref[0]
    sq_all = jnp.sum(xt * xt, axis=0, keepdims=True)
    mm = jax.lax.dot_general(
        xr, xt, (((1,), (0,)), ((), ())),
        preferred_element_type=jnp.float32)
    sq_r = jnp.sum(xr * xr, axis=1, keepdims=True)
    dist_ref[0] = sq_r + sq_all - 2.0 * mm
    row0 = rb * BR + n * M
    dst_ref[0] = lax.broadcasted_iota(jnp.int32, (BR, K), 0) + row0


def _merge16(bk, bi, ck, ci):
    sk, si = lax.sort((ck, ci), dimension=0, num_keys=1)
    rk = lax.rev(bk, (0,))
    ri = lax.rev(bi, (0,))
    tk = jnp.minimum(sk, rk)
    ti = jnp.where(sk <= rk, si, ri)
    ok, oi = lax.sort((tk, ti), dimension=0, num_keys=1)
    return ok, oi


def _make_sc_topk(R, M):
    info = plsc.get_sparse_core_info()
    nw = info.num_cores * info.num_subcores
    rows_per_w = R // nw
    nblocks = rows_per_w // RB
    nchunk = M // 16

    mesh = plsc.VectorSubcoreMesh(core_axis_name="c", subcore_axis_name="s")

    @functools.partial(
        pl.kernel, mesh=mesh,
        out_type=jax.ShapeDtypeStruct((R, K), jnp.int32),
        compiler_params=pltpu.CompilerParams(needs_layout_passes=False),
        scratch_types=[
            pltpu.VMEM((RB, M), jnp.float32),
            pltpu.VMEM((RB, K), jnp.int32),
        ],
    )
    def sc_topk(dist_hbm, out_hbm, buf, obuf):
        wid = lax.axis_index("s") * info.num_cores + lax.axis_index("c")
        iota16 = lax.broadcasted_iota(jnp.int32, (16,), 0)
        inf16 = jnp.full((16,), jnp.inf, jnp.float32)
        zero16 = jnp.zeros((16,), jnp.int32)

        G = 16

        def block_body(b, _):
            base = wid * rows_per_w + b * RB
            pltpu.sync_copy(dist_hbm.at[pl.ds(base, RB)], buf)

            for g in range(RB // G):
                def chunk_body(c, carry):
                    ci = iota16 + c * 16
                    outs = []
                    for i in range(G):
                        bk, bi = carry[2 * i], carry[2 * i + 1]
                        ck = buf[g * G + i, pl.ds(c * 16, 16)]
                        nk, ni = _merge16(bk, bi, ck, ci)
                        outs += [nk, ni]
                    return tuple(outs)

                res = lax.fori_loop(
                    0, nchunk, chunk_body, (inf16, zero16) * G)
                for i in range(G):
                    row = base + g * G + i
                    off = (row // M) * M
                    obuf[g * G + i, :] = res[2 * i + 1] + off

            pltpu.sync_copy(obuf, out_hbm.at[pl.ds(base, RB)])
            return ()

        lax.fori_loop(0, nblocks, block_body, ())

    return sc_topk


@jax.jit
def kernel(x):
    if x.ndim == 2:
        x = x[None, :, :]
    N, M, D = x.shape
    xt = x.transpose(0, 2, 1)

    grid = (N, M // BR)
    dist, dst = pl.pallas_call(
        functools.partial(_dist_body, M=M, K=K),
        grid=grid,
        in_specs=[
            pl.BlockSpec((1, BR, D), lambda n, r: (n, r, 0)),
            pl.BlockSpec((1, D, M), lambda n, r: (n, 0, 0)),
        ],
        out_specs=[
            pl.BlockSpec((1, BR, M), lambda n, r: (n, r, 0)),
            pl.BlockSpec((1, BR, K), lambda n, r: (n, r, 0)),
        ],
        out_shape=[
            jax.ShapeDtypeStruct((N, M, M), jnp.float32),
            jax.ShapeDtypeStruct((N, M, K), jnp.int32),
        ],
    )(x, xt)

    sc_topk = _make_sc_topk(N * M, M)
    src = sc_topk(dist.reshape(N * M, M))
    return src.reshape(-1), dst.reshape(-1)


# speedup vs baseline: 1.0001x; 1.0001x over previous
import functools

import jax
import jax.numpy as jnp
from jax import lax
from jax.experimental import pallas as pl
from jax.experimental.pallas import tpu as pltpu
from jax.experimental.pallas import tpu_sc as plsc

K = 16
BR = 256
RB = 16


def _dist_body(xr_ref, xt_ref, dist_ref, dst_ref, *, M, K):
    n = pl.program_id(0)
    rb = pl.program_id(1)
    xr = xr_ref[0]
    xt = xt_ref[0]
    sq_all = jnp.sum(xt * xt, axis=0, keepdims=True)
    mm = jax.lax.dot_general(
        xr, xt, (((1,), (0,)), ((), ())),
        preferred_element_type=jnp.float32)
    sq_r = jnp.sum(xr * xr, axis=1, keepdims=True)
    dist_ref[0] = sq_r + sq_all - 2.0 * mm
    row0 = rb * BR + n * M
    dst_ref[0] = lax.broadcasted_iota(jnp.int32, (BR, K), 0) + row0


def _merge16(bk, bi, ck, ci):
    sk, si = lax.sort((ck, ci), dimension=0, num_keys=1)
    rk = lax.rev(bk, (0,))
    ri = lax.rev(bi, (0,))
    tk = jnp.minimum(sk, rk)
    ti = jnp.where(sk <= rk, si, ri)
    ok, oi = lax.sort((tk, ti), dimension=0, num_keys=1)
    return ok, oi


def _make_sc_topk(R, M):
    info = plsc.get_sparse_core_info()
    nw = info.num_cores * info.num_subcores
    rows_per_w = R // nw
    nblocks = rows_per_w // RB
    nchunk = M // 16

    mesh = plsc.VectorSubcoreMesh(core_axis_name="c", subcore_axis_name="s")

    @functools.partial(
        pl.kernel, mesh=mesh,
        out_type=jax.ShapeDtypeStruct((R, K), jnp.int32),
        compiler_params=pltpu.CompilerParams(needs_layout_passes=False),
        scratch_types=[
            pltpu.VMEM((RB, M), jnp.float32),
            pltpu.VMEM((RB, K), jnp.int32),
        ],
    )
    def sc_topk(dist_hbm, out_hbm, buf, obuf):
        wid = lax.axis_index("s") * info.num_cores + lax.axis_index("c")
        iota16 = lax.broadcasted_iota(jnp.int32, (16,), 0)
        inf16 = jnp.full((16,), jnp.inf, jnp.float32)
        zero16 = jnp.zeros((16,), jnp.int32)

        G = 8

        def block_body(b, _):
            base = wid * rows_per_w + b * RB
            pltpu.sync_copy(dist_hbm.at[pl.ds(base, RB)], buf)

            for g in range(RB // G):
                def chunk_body(c, carry):
                    ci = iota16 + c * 16
                    outs = []
                    for i in range(G):
                        bk, bi = carry[2 * i], carry[2 * i + 1]
                        ck = buf[g * G + i, pl.ds(c * 16, 16)]
                        nk, ni = _merge16(bk, bi, ck, ci)
                        outs += [nk, ni]
                    return tuple(outs)

                res = lax.fori_loop(
                    0, nchunk, chunk_body, (inf16, zero16) * G)
                for i in range(G):
                    row = base + g * G + i
                    off = (row // M) * M
                    obuf[g * G + i, :] = res[2 * i + 1] + off

            pltpu.sync_copy(obuf, out_hbm.at[pl.ds(base, RB)])
            return ()

        lax.fori_loop(0, nblocks, block_body, ())

    return sc_topk


@jax.jit
def kernel(x):
    if x.ndim == 2:
        x = x[None, :, :]
    N, M, D = x.shape
    xt = x.transpose(0, 2, 1)

    grid = (N, M // BR)
    dist, dst = pl.pallas_call(
        functools.partial(_dist_body, M=M, K=K),
        grid=grid,
        in_specs=[
            pl.BlockSpec((1, BR, D), lambda n, r: (n, r, 0)),
            pl.BlockSpec((1, D, M), lambda n, r: (n, 0, 0)),
        ],
        out_specs=[
            pl.BlockSpec((1, BR, M), lambda n, r: (n, r, 0)),
            pl.BlockSpec((1, BR, K), lambda n, r: (n, r, 0)),
        ],
        out_shape=[
            jax.ShapeDtypeStruct((N, M, M), jnp.float32),
            jax.ShapeDtypeStruct((N, M, K), jnp.int32),
        ],
    )(x, xt)

    sc_topk = _make_sc_topk(N * M, M)
    src = sc_topk(dist.reshape(N * M, M))
    return src.reshape(-1), dst.reshape(-1)


# speedup vs baseline: 1.0876x; 1.0875x over previous
import functools

import jax
import jax.numpy as jnp
from jax import lax
from jax.experimental import pallas as pl
from jax.experimental.pallas import tpu as pltpu
from jax.experimental.pallas import tpu_sc as plsc

K = 16
BR = 256
RB = 32


def _dist_body(xr_ref, xt_ref, dist_ref, dst_ref, *, M, K):
    n = pl.program_id(0)
    rb = pl.program_id(1)
    xr = xr_ref[0]
    xt = xt_ref[0]
    sq_all = jnp.sum(xt * xt, axis=0, keepdims=True)
    mm = jax.lax.dot_general(
        xr, xt, (((1,), (0,)), ((), ())),
        preferred_element_type=jnp.float32)
    sq_r = jnp.sum(xr * xr, axis=1, keepdims=True)
    dist_ref[0] = sq_r + sq_all - 2.0 * mm
    row0 = rb * BR + n * M
    dst_ref[0] = lax.broadcasted_iota(jnp.int32, (BR, K), 0) + row0


def _merge16(bk, bi, ck, ci):
    sk, si = plsc.sort_key_val(ck, ci)
    tk = jnp.minimum(sk, bk)
    ti = jnp.where(sk <= bk, si, bi)
    ok, oi = plsc.sort_key_val(tk, ti, descending=True)
    return ok, oi


def _make_sc_topk(R, M):
    info = plsc.get_sparse_core_info()
    nw = info.num_cores * info.num_subcores
    rows_per_w = R // nw
    nblocks = rows_per_w // RB
    nchunk = M // 16

    mesh = plsc.VectorSubcoreMesh(core_axis_name="c", subcore_axis_name="s")

    @functools.partial(
        pl.kernel, mesh=mesh,
        out_type=jax.ShapeDtypeStruct((R, K), jnp.int32),
        compiler_params=pltpu.CompilerParams(needs_layout_passes=False),
        scratch_types=[
            pltpu.VMEM((RB, M), jnp.float32),
            pltpu.VMEM((RB, K), jnp.int32),
        ],
    )
    def sc_topk(dist_hbm, out_hbm, buf, obuf):
        wid = lax.axis_index("s") * info.num_cores + lax.axis_index("c")
        iota16 = lax.broadcasted_iota(jnp.int32, (16,), 0)
        inf16 = jnp.full((16,), jnp.inf, jnp.float32)
        zero16 = jnp.zeros((16,), jnp.int32)

        G = 8

        def block_body(b, _):
            base = wid * rows_per_w + b * RB
            pltpu.sync_copy(dist_hbm.at[pl.ds(base, RB)], buf)

            for g in range(RB // G):
                def chunk_body(c, carry):
                    ci = iota16 + c * 16
                    outs = []
                    for i in range(G):
                        bk, bi = carry[2 * i], carry[2 * i + 1]
                        ck = buf[g * G + i, pl.ds(c * 16, 16)]
                        nk, ni = _merge16(bk, bi, ck, ci)
                        outs += [nk, ni]
                    return tuple(outs)

                res = lax.fori_loop(
                    0, nchunk, chunk_body, (inf16, zero16) * G)
                for i in range(G):
                    row = base + g * G + i
                    off = (row // M) * M
                    obuf[g * G + i, :] = lax.rev(res[2 * i + 1], (0,)) + off

            pltpu.sync_copy(obuf, out_hbm.at[pl.ds(base, RB)])
            return ()

        lax.fori_loop(0, nblocks, block_body, ())

    return sc_topk


@jax.jit
def kernel(x):
    if x.ndim == 2:
        x = x[None, :, :]
    N, M, D = x.shape
    xt = x.transpose(0, 2, 1)

    grid = (N, M // BR)
    dist, dst = pl.pallas_call(
        functools.partial(_dist_body, M=M, K=K),
        grid=grid,
        in_specs=[
            pl.BlockSpec((1, BR, D), lambda n, r: (n, r, 0)),
            pl.BlockSpec((1, D, M), lambda n, r: (n, 0, 0)),
        ],
        out_specs=[
            pl.BlockSpec((1, BR, M), lambda n, r: (n, r, 0)),
            pl.BlockSpec((1, BR, K), lambda n, r: (n, r, 0)),
        ],
        out_shape=[
            jax.ShapeDtypeStruct((N, M, M), jnp.float32),
            jax.ShapeDtypeStruct((N, M, K), jnp.int32),
        ],
    )(x, xt)

    sc_topk = _make_sc_topk(N * M, M)
    src = sc_topk(dist.reshape(N * M, M))
    return src.reshape(-1), dst.reshape(-1)
